# baseline (speedup 1.0000x reference)
import functools

import numpy as np
import jax
import jax.numpy as jnp
from jax import lax
from jax.experimental import pallas as pl
from jax.experimental.pallas import tpu as pltpu
from jax.experimental.pallas import tpu_sc as plsc

VOCAB = 30522
EMBED = 128
HIDDEN = 64
NUM_CLASSES = 4
B = 4096
L = 200

LANES = 16
CHUNK = 100
CHUNKS_PER_ROW = L // CHUNK
WORDS = EMBED // 2
NGROUP = WORDS // LANES

_PERM = np.array(
    [32 * (k // 2) + 2 * j + (k % 2) for k in range(8) for j in range(16)],
    dtype=np.int32,
)


def _make_pool_kernel():
    info = plsc.get_sparse_core_info()
    nw = info.num_cores * info.num_subcores
    rows_per_w = B // nw
    chunks_per_w = rows_per_w * CHUNKS_PER_ROW

    mesh = plsc.VectorSubcoreMesh(core_axis_name="c", subcore_axis_name="s")

    @functools.partial(
        pl.kernel,
        out_type=jax.ShapeDtypeStruct((B, EMBED), jnp.float32),
        mesh=mesh,
        scratch_types=[
            pltpu.VMEM((chunks_per_w, CHUNK), jnp.int32),
            pltpu.VMEM((2, CHUNK, WORDS), jnp.uint32),
            pltpu.VMEM((rows_per_w, EMBED), jnp.float32),
            pltpu.SemaphoreType.DMA,
            pltpu.SemaphoreType.DMA,
        ],
        compiler_params=pltpu.CompilerParams(use_tc_tiling_on_sc=False),
    )
    def pool(ids_hbm, table_hbm, out_hbm, idx_v, rows_v, pooled_v,
             sem0, sem1):
        cid = lax.axis_index("c")
        sid = lax.axis_index("s")
        wid = sid * info.num_cores + cid

        pltpu.sync_copy(ids_hbm.at[pl.ds(wid * chunks_per_w, chunks_per_w)],
                        idx_v)

        inv_l = jnp.float32(1.0 / L)
        sems = (sem0, sem1)
        himask = jnp.full((LANES,), 0xFFFF0000, jnp.uint32)
        shift = jnp.full((LANES,), 16, jnp.uint32)

        def fire(k, buf):
            pltpu.async_copy(table_hbm.at[idx_v.at[k]], rows_v.at[buf],
                             sems[buf])

        def drain_reduce(buf, acc):
            pltpu.make_async_copy(table_hbm.at[idx_v.at[0]], rows_v.at[buf],
                                  sems[buf]).wait()

            @plsc.parallel_loop(0, CHUNK, unroll=4, carry=acc)
            def red_body(l, acc):
                acc = list(acc)
                for c in range(NGROUP):
                    w = rows_v[buf, l, pl.ds(c * LANES, LANES)]
                    lo = lax.bitcast_convert_type(w << shift, jnp.float32)
                    hi = lax.bitcast_convert_type(w & himask, jnp.float32)
                    acc[2 * c] = acc[2 * c] + lo
                    acc[2 * c + 1] = acc[2 * c + 1] + hi
                return tuple(acc)

            return red_body

        zeros = (jnp.zeros((LANES,), jnp.float32),) * (2 * NGROUP)

        fire(0, 0)

        def row_body(b, _):
            fire(2 * b + 1, 1)
            acc = drain_reduce(0, zeros)
            fire(2 * b + 2, 0)
            acc = drain_reduce(1, acc)
            for k in range(2 * NGROUP):
                pooled_v[b, pl.ds(k * LANES, LANES)] = acc[k] * inv_l
            return 0

        lax.fori_loop(0, rows_per_w - 1, row_body, 0)

        b_last = rows_per_w - 1
        fire(2 * b_last + 1, 1)
        acc = drain_reduce(0, zeros)
        acc = drain_reduce(1, acc)
        for k in range(2 * NGROUP):
            pooled_v[b_last, pl.ds(k * LANES, LANES)] = acc[k] * inv_l

        pltpu.sync_copy(pooled_v, out_hbm.at[pl.ds(wid * rows_per_w,
                                                   rows_per_w)])

    return pool


def _mlp_body(x_ref, w1_ref, b1_ref, w2_ref, b2_ref, o_ref):
    h = jnp.dot(x_ref[...], w1_ref[...], preferred_element_type=jnp.float32)
    h = jnp.maximum(h + b1_ref[...], 0.0)
    o = jnp.dot(h, w2_ref[...], preferred_element_type=jnp.float32)
    o_ref[...] = o + b2_ref[...]


@jax.jit
def kernel(input_ids, attention_mask, emb, W1, b1, W2, b2):
    del attention_mask
    ids2 = input_ids.astype(jnp.int32).reshape(B * CHUNKS_PER_ROW, CHUNK)
    u = lax.bitcast_convert_type(emb, jnp.uint32)
    rtne = lambda x: x + jnp.uint32(0x7FFF) + ((x >> 16) & jnp.uint32(1))
    table = ((rtne(u[:, 0::2]) >> 16)
             | (rtne(u[:, 1::2]) & jnp.uint32(0xFFFF0000)))
    pooled = _make_pool_kernel()(ids2, table)
    w1p = W1[jnp.asarray(_PERM)]
    out = pl.pallas_call(
        _mlp_body,
        out_shape=jax.ShapeDtypeStruct((B, NUM_CLASSES), jnp.float32),
    )(pooled, w1p, b1.reshape(1, HIDDEN), W2, b2.reshape(1, NUM_CLASSES))
    return out


# speedup vs baseline: 3.3957x; 3.3957x over previous
import functools

import numpy as np
import jax
import jax.numpy as jnp
from jax import lax
from jax.experimental import pallas as pl
from jax.experimental.pallas import tpu as pltpu
from jax.experimental.pallas import tpu_sc as plsc

VOCAB = 30522
EMBED = 128
HIDDEN = 64
NUM_CLASSES = 4
B = 4096
L = 200

LANES = 16
CHUNK = 100
CHUNKS_PER_ROW = L // CHUNK
WORDS = EMBED // 2
NGROUP = WORDS // LANES

_PERM = np.array(
    [(k % 2) * 64 + 16 * (k // 2) + j for k in range(8) for j in range(16)],
    dtype=np.int32,
)


def _make_pool_kernel():
    info = plsc.get_sparse_core_info()
    nw = info.num_cores * info.num_subcores
    rows_per_w = B // nw
    chunks_per_w = rows_per_w * CHUNKS_PER_ROW

    mesh = plsc.VectorSubcoreMesh(core_axis_name="c", subcore_axis_name="s")

    @functools.partial(
        pl.kernel,
        out_type=jax.ShapeDtypeStruct((B, EMBED), jnp.float32),
        mesh=mesh,
        scratch_types=[
            pltpu.VMEM((chunks_per_w, CHUNK), jnp.int32),
            pltpu.VMEM((2, CHUNK, WORDS), jnp.uint32),
            pltpu.VMEM((rows_per_w, EMBED), jnp.float32),
            pltpu.SemaphoreType.DMA,
            pltpu.SemaphoreType.DMA,
        ],
        compiler_params=pltpu.CompilerParams(use_tc_tiling_on_sc=False),
    )
    def pool(ids_hbm, table_hbm, out_hbm, idx_v, rows_v, pooled_v,
             sem0, sem1):
        cid = lax.axis_index("c")
        sid = lax.axis_index("s")
        wid = sid * info.num_cores + cid

        pltpu.sync_copy(ids_hbm.at[pl.ds(wid * chunks_per_w, chunks_per_w)],
                        idx_v)

        inv_l = jnp.float32(1.0 / L)
        sems = (sem0, sem1)
        himask = jnp.full((LANES,), 0xFFFF0000, jnp.uint32)
        shift = jnp.full((LANES,), 16, jnp.uint32)

        def fire(k, buf):
            pltpu.async_copy(table_hbm.at[idx_v.at[k]], rows_v.at[buf],
                             sems[buf])

        def drain_reduce(buf, acc):
            pltpu.make_async_copy(table_hbm.at[idx_v.at[0]], rows_v.at[buf],
                                  sems[buf]).wait()

            @plsc.parallel_loop(0, CHUNK, unroll=4, carry=acc)
            def red_body(l, acc):
                acc = list(acc)
                for c in range(NGROUP):
                    w = rows_v[buf, l, pl.ds(c * LANES, LANES)]
                    lo = lax.bitcast_convert_type(w << shift, jnp.float32)
                    hi = lax.bitcast_convert_type(w & himask, jnp.float32)
                    acc[2 * c] = acc[2 * c] + lo
                    acc[2 * c + 1] = acc[2 * c + 1] + hi
                return tuple(acc)

            return red_body

        zeros = (jnp.zeros((LANES,), jnp.float32),) * (2 * NGROUP)

        fire(0, 0)

        def row_body(b, _):
            fire(2 * b + 1, 1)
            acc = drain_reduce(0, zeros)
            fire(2 * b + 2, 0)
            acc = drain_reduce(1, acc)
            for k in range(2 * NGROUP):
                pooled_v[b, pl.ds(k * LANES, LANES)] = acc[k] * inv_l
            return 0

        lax.fori_loop(0, rows_per_w - 1, row_body, 0)

        b_last = rows_per_w - 1
        fire(2 * b_last + 1, 1)
        acc = drain_reduce(0, zeros)
        acc = drain_reduce(1, acc)
        for k in range(2 * NGROUP):
            pooled_v[b_last, pl.ds(k * LANES, LANES)] = acc[k] * inv_l

        pltpu.sync_copy(pooled_v, out_hbm.at[pl.ds(wid * rows_per_w,
                                                   rows_per_w)])

    return pool


def _mlp_body(x_ref, w1_ref, b1_ref, w2_ref, b2_ref, o_ref):
    h = jnp.dot(x_ref[...], w1_ref[...], preferred_element_type=jnp.float32)
    h = jnp.maximum(h + b1_ref[...], 0.0)
    o = jnp.dot(h, w2_ref[...], preferred_element_type=jnp.float32)
    o_ref[...] = o + b2_ref[...]


@jax.jit
def kernel(input_ids, attention_mask, emb, W1, b1, W2, b2):
    del attention_mask
    ids2 = input_ids.astype(jnp.int32).reshape(B * CHUNKS_PER_ROW, CHUNK)
    u = lax.bitcast_convert_type(emb, jnp.uint32)
    rtne = lambda x: x + jnp.uint32(0x7FFF) + ((x >> 16) & jnp.uint32(1))
    table = ((rtne(u[:, :WORDS]) >> 16)
             | (rtne(u[:, WORDS:]) & jnp.uint32(0xFFFF0000)))
    pooled = _make_pool_kernel()(ids2, table)
    w1p = W1[jnp.asarray(_PERM)]
    out = pl.pallas_call(
        _mlp_body,
        out_shape=jax.ShapeDtypeStruct((B, NUM_CLASSES), jnp.float32),
    )(pooled, w1p, b1.reshape(1, HIDDEN), W2, b2.reshape(1, NUM_CLASSES))
    return out


# speedup vs baseline: 3.4144x; 1.0055x over previous
import functools

import numpy as np
import jax
import jax.numpy as jnp
from jax import lax
from jax.experimental import pallas as pl
from jax.experimental.pallas import tpu as pltpu
from jax.experimental.pallas import tpu_sc as plsc

VOCAB = 30522
EMBED = 128
HIDDEN = 64
NUM_CLASSES = 4
B = 4096
L = 200

LANES = 16
CH = (96, 104)
OFF = (0, 96)
WORDS = EMBED // 2
NGROUP = WORDS // LANES

_PERM = np.array(
    [(k % 2) * 64 + 16 * (k // 2) + j for k in range(8) for j in range(16)],
    dtype=np.int32,
)


def _make_pool_kernel():
    info = plsc.get_sparse_core_info()
    nw = info.num_cores * info.num_subcores
    rows_per_w = B // nw

    mesh = plsc.VectorSubcoreMesh(core_axis_name="c", subcore_axis_name="s")

    @functools.partial(
        pl.kernel,
        out_type=jax.ShapeDtypeStruct((B, EMBED), jnp.float32),
        mesh=mesh,
        scratch_types=[
            pltpu.VMEM((rows_per_w, L), jnp.int32),
            pltpu.VMEM((2, max(CH), WORDS), jnp.uint32),
            pltpu.VMEM((rows_per_w, EMBED), jnp.float32),
            pltpu.SemaphoreType.DMA,
            pltpu.SemaphoreType.DMA,
        ],
        compiler_params=pltpu.CompilerParams(use_tc_tiling_on_sc=False),
    )
    def pool(ids_hbm, table_hbm, out_hbm, idx_v, rows_v, pooled_v,
             sem0, sem1):
        cid = lax.axis_index("c")
        sid = lax.axis_index("s")
        wid = sid * info.num_cores + cid

        pltpu.sync_copy(ids_hbm.at[pl.ds(wid * rows_per_w, rows_per_w)],
                        idx_v)

        inv_l = jnp.float32(1.0 / L)
        sems = (sem0, sem1)
        shift = jnp.full((LANES,), 16, jnp.uint32)

        def fire(b, j, buf):
            pltpu.async_copy(
                table_hbm.at[idx_v.at[b, pl.ds(OFF[j], CH[j])]],
                rows_v.at[buf, pl.ds(0, CH[j])], sems[buf])

        def drain_reduce(buf, acc):
            pltpu.make_async_copy(
                table_hbm.at[idx_v.at[0, pl.ds(OFF[buf], CH[buf])]],
                rows_v.at[buf, pl.ds(0, CH[buf])], sems[buf]).wait()

            @plsc.parallel_loop(0, CH[buf], unroll=4, carry=acc)
            def red_body(l, acc):
                acc = list(acc)
                for c in range(NGROUP):
                    w = rows_v[buf, l, pl.ds(c * LANES, LANES)]
                    lo = lax.bitcast_convert_type(w << shift, jnp.float32)
                    hi = lax.bitcast_convert_type(w, jnp.float32)
                    acc[2 * c] = acc[2 * c] + lo
                    acc[2 * c + 1] = acc[2 * c + 1] + hi
                return tuple(acc)

            return red_body

        zeros = (jnp.zeros((LANES,), jnp.float32),) * (2 * NGROUP)

        fire(0, 0, 0)

        def row_body(b, _):
            fire(b, 1, 1)
            acc = drain_reduce(0, zeros)
            fire(b + 1, 0, 0)
            acc = drain_reduce(1, acc)
            for k in range(2 * NGROUP):
                pooled_v[b, pl.ds(k * LANES, LANES)] = acc[k] * inv_l
            return 0

        lax.fori_loop(0, rows_per_w - 1, row_body, 0)

        b_last = rows_per_w - 1
        fire(b_last, 1, 1)
        acc = drain_reduce(0, zeros)
        acc = drain_reduce(1, acc)
        for k in range(2 * NGROUP):
            pooled_v[b_last, pl.ds(k * LANES, LANES)] = acc[k] * inv_l

        pltpu.sync_copy(pooled_v, out_hbm.at[pl.ds(wid * rows_per_w,
                                                   rows_per_w)])

    return pool


def _mlp_body(x_ref, w1_ref, b1_ref, w2_ref, b2_ref, o_ref):
    h = jnp.dot(x_ref[...], w1_ref[...], preferred_element_type=jnp.float32)
    h = jnp.maximum(h + b1_ref[...], 0.0)
    o = jnp.dot(h, w2_ref[...], preferred_element_type=jnp.float32)
    o_ref[...] = o + b2_ref[...]


@jax.jit
def kernel(input_ids, attention_mask, emb, W1, b1, W2, b2):
    del attention_mask
    ids2 = input_ids.astype(jnp.int32)
    u = lax.bitcast_convert_type(emb, jnp.uint32)
    rtne = lambda x: x + jnp.uint32(0x7FFF) + ((x >> 16) & jnp.uint32(1))
    table = ((rtne(u[:, :WORDS]) >> 16)
             | (rtne(u[:, WORDS:]) & jnp.uint32(0xFFFF0000)))
    pooled = _make_pool_kernel()(ids2, table)
    w1p = W1[jnp.asarray(_PERM)]
    out = pl.pallas_call(
        _mlp_body,
        out_shape=jax.ShapeDtypeStruct((B, NUM_CLASSES), jnp.float32),
    )(pooled, w1p, b1.reshape(1, HIDDEN), W2, b2.reshape(1, NUM_CLASSES))
    return out


# speedup vs baseline: 3.4217x; 1.0021x over previous
import functools

import numpy as np
import jax
import jax.numpy as jnp
from jax import lax
from jax.experimental import pallas as pl
from jax.experimental.pallas import tpu as pltpu
from jax.experimental.pallas import tpu_sc as plsc

VOCAB = 30522
EMBED = 128
HIDDEN = 64
NUM_CLASSES = 4
B = 4096
L = 200

LANES = 16
CH = (96, 104)
OFF = (0, 96)
WORDS = EMBED // 2
NGROUP = WORDS // LANES

_PERM = np.array(
    [(k % 2) * 64 + 16 * (k // 2) + j for k in range(8) for j in range(16)],
    dtype=np.int32,
)


def _make_pool_kernel():
    info = plsc.get_sparse_core_info()
    nw = info.num_cores * info.num_subcores
    rows_per_w = B // nw

    mesh = plsc.VectorSubcoreMesh(core_axis_name="c", subcore_axis_name="s")

    @functools.partial(
        pl.kernel,
        out_type=jax.ShapeDtypeStruct((B, EMBED), jnp.float32),
        mesh=mesh,
        scratch_types=[
            pltpu.VMEM((rows_per_w * L,), jnp.int32),
            pltpu.VMEM((2, max(CH), WORDS), jnp.uint32),
            pltpu.VMEM((rows_per_w, EMBED), jnp.float32),
            pltpu.SemaphoreType.DMA,
            pltpu.SemaphoreType.DMA,
        ],
        compiler_params=pltpu.CompilerParams(use_tc_tiling_on_sc=False),
    )
    def pool(ids_hbm, table_hbm, out_hbm, idx_v, rows_v, pooled_v,
             sem0, sem1):
        cid = lax.axis_index("c")
        sid = lax.axis_index("s")
        wid = sid * info.num_cores + cid

        n_ids = rows_per_w * L
        pltpu.sync_copy(ids_hbm.at[pl.ds(wid * n_ids, n_ids)], idx_v)

        sems = (sem0, sem1)
        shift = jnp.full((LANES,), 16, jnp.uint32)
        himask = jnp.full((LANES,), 0xFFFF0000, jnp.uint32)

        def fire(b, j, buf):
            pltpu.async_copy(
                table_hbm.at[idx_v.at[pl.ds(b * L + OFF[j], CH[j])]],
                rows_v.at[buf, pl.ds(0, CH[j])], sems[buf])

        def drain_reduce(buf, acc):
            pltpu.make_async_copy(
                table_hbm.at[idx_v.at[pl.ds(OFF[buf], CH[buf])]],
                rows_v.at[buf, pl.ds(0, CH[buf])], sems[buf]).wait()

            @plsc.parallel_loop(0, CH[buf], unroll=4, carry=acc)
            def red_body(l, acc):
                acc = list(acc)
                for c in range(NGROUP):
                    w = rows_v[buf, l, pl.ds(c * LANES, LANES)]
                    lo = lax.bitcast_convert_type(w << shift, jnp.float32)
                    hi = lax.bitcast_convert_type(w & himask, jnp.float32)
                    acc[2 * c] = acc[2 * c] + lo
                    acc[2 * c + 1] = acc[2 * c + 1] + hi
                return tuple(acc)

            return red_body

        zeros = (jnp.zeros((LANES,), jnp.float32),) * (2 * NGROUP)

        fire(0, 0, 0)

        def row_body(b, _):
            fire(b, 1, 1)
            acc = drain_reduce(0, zeros)
            fire(b + 1, 0, 0)
            acc = drain_reduce(1, acc)
            for k in range(2 * NGROUP):
                pooled_v[b, pl.ds(k * LANES, LANES)] = acc[k]
            return 0

        lax.fori_loop(0, rows_per_w - 1, row_body, 0)

        b_last = rows_per_w - 1
        fire(b_last, 1, 1)
        acc = drain_reduce(0, zeros)
        acc = drain_reduce(1, acc)
        for k in range(2 * NGROUP):
            pooled_v[b_last, pl.ds(k * LANES, LANES)] = acc[k]

        pltpu.sync_copy(pooled_v, out_hbm.at[pl.ds(wid * rows_per_w,
                                                   rows_per_w)])

    return pool


def _mlp_body(x_ref, w1_ref, b1_ref, w2_ref, b2_ref, o_ref):
    h = jnp.dot(x_ref[...], w1_ref[...], preferred_element_type=jnp.float32)
    h = jnp.maximum(h + b1_ref[...], 0.0)
    o = jnp.dot(h, w2_ref[...], preferred_element_type=jnp.float32)
    o_ref[...] = o + b2_ref[...]


@jax.jit
def kernel(input_ids, attention_mask, emb, W1, b1, W2, b2):
    del attention_mask
    ids2 = input_ids.astype(jnp.int32).reshape(B * L)
    u = lax.bitcast_convert_type(emb, jnp.uint32)
    rtne = lambda x: x + jnp.uint32(0x7FFF) + ((x >> 16) & jnp.uint32(1))
    table = ((rtne(u[:, :WORDS]) >> 16)
             | (rtne(u[:, WORDS:]) & jnp.uint32(0xFFFF0000)))
    pooled = _make_pool_kernel()(ids2, table)
    w1p = W1[jnp.asarray(_PERM)] * jnp.float32(1.0 / L)
    out = pl.pallas_call(
        _mlp_body,
        out_shape=jax.ShapeDtypeStruct((B, NUM_CLASSES), jnp.float32),
    )(pooled, w1p, b1.reshape(1, HIDDEN), W2, b2.reshape(1, NUM_CLASSES))
    return out


# speedup vs baseline: 3.4232x; 1.0004x over previous
import functools

import numpy as np
import jax
import jax.numpy as jnp
from jax import lax
from jax.experimental import pallas as pl
from jax.experimental.pallas import tpu as pltpu
from jax.experimental.pallas import tpu_sc as plsc

VOCAB = 30522
EMBED = 128
HIDDEN = 64
NUM_CLASSES = 4
B = 4096
L = 200

LANES = 16
CH = (96, 104)
OFF = (0, 96)
WORDS = EMBED // 2
NGROUP = WORDS // LANES

_PERM = np.array(
    [(k % 2) * 64 + 16 * (k // 2) + j for k in range(8) for j in range(16)],
    dtype=np.int32,
)


def _make_pool_kernel():
    info = plsc.get_sparse_core_info()
    nw = info.num_cores * info.num_subcores
    rows_per_w = B // nw

    mesh = plsc.VectorSubcoreMesh(core_axis_name="c", subcore_axis_name="s")

    @functools.partial(
        pl.kernel,
        out_type=jax.ShapeDtypeStruct((B, EMBED), jnp.float32),
        mesh=mesh,
        scratch_types=[
            pltpu.VMEM((rows_per_w * L,), jnp.int32),
            pltpu.VMEM((2, max(CH), WORDS), jnp.uint32),
            pltpu.VMEM((rows_per_w, EMBED), jnp.float32),
            pltpu.SemaphoreType.DMA,
            pltpu.SemaphoreType.DMA,
        ],
        compiler_params=pltpu.CompilerParams(use_tc_tiling_on_sc=False),
    )
    def pool(ids_hbm, table_hbm, out_hbm, idx_v, rows_v, pooled_v,
             sem0, sem1):
        cid = lax.axis_index("c")
        sid = lax.axis_index("s")
        wid = sid * info.num_cores + cid

        n_ids = rows_per_w * L
        pltpu.sync_copy(ids_hbm.at[pl.ds(wid * n_ids, n_ids)], idx_v)

        sems = (sem0, sem1)
        shift = jnp.full((LANES,), 16, jnp.uint32)
        himask = jnp.full((LANES,), 0xFFFF0000, jnp.uint32)

        def fire(b, j, buf):
            pltpu.async_copy(
                table_hbm.at[idx_v.at[pl.ds(b * L + OFF[j], CH[j])]],
                rows_v.at[buf, pl.ds(0, CH[j])], sems[buf])

        def drain_reduce(buf, acc):
            pltpu.make_async_copy(
                table_hbm.at[idx_v.at[pl.ds(OFF[buf], CH[buf])]],
                rows_v.at[buf, pl.ds(0, CH[buf])], sems[buf]).wait()

            @plsc.parallel_loop(0, CH[buf], unroll=8, carry=acc)
            def red_body(l, acc):
                acc = list(acc)
                for c in range(NGROUP):
                    w = rows_v[buf, l, pl.ds(c * LANES, LANES)]
                    lo = lax.bitcast_convert_type(w << shift, jnp.float32)
                    hi = lax.bitcast_convert_type(w & himask, jnp.float32)
                    acc[2 * c] = acc[2 * c] + lo
                    acc[2 * c + 1] = acc[2 * c + 1] + hi
                return tuple(acc)

            return red_body

        zeros = (jnp.zeros((LANES,), jnp.float32),) * (2 * NGROUP)

        fire(0, 0, 0)

        def row_body(b, _):
            fire(b, 1, 1)
            acc = drain_reduce(0, zeros)
            fire(b + 1, 0, 0)
            acc = drain_reduce(1, acc)
            for k in range(2 * NGROUP):
                pooled_v[b, pl.ds(k * LANES, LANES)] = acc[k]
            return 0

        lax.fori_loop(0, rows_per_w - 1, row_body, 0)

        b_last = rows_per_w - 1
        fire(b_last, 1, 1)
        acc = drain_reduce(0, zeros)
        acc = drain_reduce(1, acc)
        for k in range(2 * NGROUP):
            pooled_v[b_last, pl.ds(k * LANES, LANES)] = acc[k]

        pltpu.sync_copy(pooled_v, out_hbm.at[pl.ds(wid * rows_per_w,
                                                   rows_per_w)])

    return pool


def _mlp_body(x_ref, w1_ref, b1_ref, w2_ref, b2_ref, o_ref):
    h = jnp.dot(x_ref[...], w1_ref[...], preferred_element_type=jnp.float32)
    h = jnp.maximum(h + b1_ref[...], 0.0)
    o = jnp.dot(h, w2_ref[...], preferred_element_type=jnp.float32)
    o_ref[...] = o + b2_ref[...]


@jax.jit
def kernel(input_ids, attention_mask, emb, W1, b1, W2, b2):
    del attention_mask
    ids2 = input_ids.astype(jnp.int32).reshape(B * L)
    u = lax.bitcast_convert_type(emb, jnp.uint32)
    rtne = lambda x: x + jnp.uint32(0x7FFF) + ((x >> 16) & jnp.uint32(1))
    table = ((rtne(u[:, :WORDS]) >> 16)
             | (rtne(u[:, WORDS:]) & jnp.uint32(0xFFFF0000)))
    pooled = _make_pool_kernel()(ids2, table)
    w1p = W1[jnp.asarray(_PERM)] * jnp.float32(1.0 / L)
    out = pl.pallas_call(
        _mlp_body,
        out_shape=jax.ShapeDtypeStruct((B, NUM_CLASSES), jnp.float32),
    )(pooled, w1p, b1.reshape(1, HIDDEN), W2, b2.reshape(1, NUM_CLASSES))
    return out
